# baseline (speedup 1.0000x reference)
import functools
import jax
import jax.numpy as jnp
from jax import lax
from jax.experimental import pallas as pl
from jax.experimental.pallas import tpu as pltpu
from jax.experimental.pallas import tpu_sc as plsc

_TOKENS = 32768
_D = 1024
_E = 8
_BT = 2048

_NC = 2
_NS = 16
_NW = _NC * _NS
_TPW = _TOKENS // _NW
_L = 16
_NCHUNK = _TPW // _L


def _tc_logits_block(x_ref, w_ref, b_ref, lt_ref):
    x = x_ref[...]
    w = w_ref[...]
    logits = jax.lax.dot_general(
        x, w, (((1,), (1,)), ((), ())), preferred_element_type=jnp.float32)
    lt_ref[...] = logits.T + b_ref[...]


_sc_mesh = plsc.VectorSubcoreMesh(core_axis_name="c", subcore_axis_name="s")


@functools.partial(
    pl.kernel,
    mesh=_sc_mesh,
    out_type=[
        jax.ShapeDtypeStruct((2, _TOKENS), jnp.float32),
        jax.ShapeDtypeStruct((2, _TOKENS), jnp.int32),
    ],
    scratch_types=[
        pltpu.VMEM((_E, _TPW), jnp.float32),
        pltpu.VMEM((2, _TPW), jnp.float32),
        pltpu.VMEM((2, _TPW), jnp.int32),
    ],
)
def _sc_select(lt_hbm, vals_hbm, idx_hbm, lbuf, vbuf, ibuf):
    wid = lax.axis_index("s") * _NC + lax.axis_index("c")
    base = wid * _TPW
    for e in range(_E):
        pltpu.sync_copy(lt_hbm.at[e, pl.ds(base, _TPW)], lbuf.at[e])

    def body(c, carry):
        vs = [lbuf[e, pl.ds(c * _L, _L)] for e in range(_E)]
        m1 = vs[0]
        i1 = jnp.zeros((_L,), jnp.int32)
        for e in range(1, _E):
            better = vs[e] > m1
            m1 = jnp.where(better, vs[e], m1)
            i1 = jnp.where(better, jnp.int32(e), i1)
        m2 = jnp.full((_L,), -jnp.inf, jnp.float32)
        i2 = jnp.full((_L,), _E, jnp.int32)
        for e in range(_E):
            v = jnp.where(i1 == e, -jnp.inf, vs[e])
            better = v > m2
            m2 = jnp.where(better, v, m2)
            i2 = jnp.where(better, jnp.int32(e), i2)
        one = jnp.float32(1.0)
        s1 = one / (one + jnp.exp(-m1))
        s2 = one / (one + jnp.exp(-m2))
        sl = pl.ds(c * _L, _L)
        vbuf[0, sl] = s1
        vbuf[1, sl] = s2
        ibuf[0, sl] = i1
        ibuf[1, sl] = i2
        return carry

    lax.fori_loop(0, _NCHUNK, body, 0)
    for r in range(2):
        pltpu.sync_copy(vbuf.at[r], vals_hbm.at[r, pl.ds(base, _TPW)])
        pltpu.sync_copy(ibuf.at[r], idx_hbm.at[r, pl.ds(base, _TPW)])


def kernel(x, W, b):
    b2 = b.reshape(_E, 1)
    grid = (_TOKENS // _BT,)
    lt = pl.pallas_call(
        _tc_logits_block,
        grid=grid,
        in_specs=[
            pl.BlockSpec((_BT, _D), lambda i: (i, 0)),
            pl.BlockSpec((_E, _D), lambda i: (0, 0)),
            pl.BlockSpec((_E, 1), lambda i: (0, 0)),
        ],
        out_specs=pl.BlockSpec((_E, _BT), lambda i: (0, i)),
        out_shape=jax.ShapeDtypeStruct((_E, _TOKENS), jnp.float32),
        compiler_params=pltpu.CompilerParams(
            dimension_semantics=("parallel",),
        ),
    )(x, W, b2)
    vals_t, idx_t = _sc_select(lt)
    return vals_t.T, idx_t.T


# speedup vs baseline: 1.5170x; 1.5170x over previous
import functools
import jax
import jax.numpy as jnp
from jax.experimental import pallas as pl
from jax.experimental.pallas import tpu as pltpu

_TOKENS = 32768
_D = 1024
_E = 8
_BT = 2048


def _gate_block(x_ref, w_ref, b_ref, vals_ref, idx_ref):
    x = x_ref[...]
    w = w_ref[...]
    logits = jax.lax.dot_general(
        x, w, (((1,), (1,)), ((), ())), preferred_element_type=jnp.float32)
    lt = logits.T + b_ref[...]

    e = jax.lax.broadcasted_iota(jnp.int32, lt.shape, 0)
    m1 = jnp.max(lt, axis=0, keepdims=True)
    i1 = jnp.min(jnp.where(lt == m1, e, _E), axis=0, keepdims=True)
    masked = jnp.where(e == i1, -jnp.inf, lt)
    m2 = jnp.max(masked, axis=0, keepdims=True)
    i2 = jnp.min(jnp.where(masked == m2, e, _E), axis=0, keepdims=True)

    vals_ref[...] = jax.nn.sigmoid(jnp.concatenate([m1, m2], axis=0))
    idx_ref[...] = jnp.concatenate([i1, i2], axis=0)


def kernel(x, W, b):
    b2 = b.reshape(_E, 1)
    grid = (_TOKENS // _BT,)
    vals_t, idx_t = pl.pallas_call(
        _gate_block,
        grid=grid,
        in_specs=[
            pl.BlockSpec((_BT, _D), lambda i: (i, 0)),
            pl.BlockSpec((_E, _D), lambda i: (0, 0)),
            pl.BlockSpec((_E, 1), lambda i: (0, 0)),
        ],
        out_specs=[
            pl.BlockSpec((2, _BT), lambda i: (0, i)),
            pl.BlockSpec((2, _BT), lambda i: (0, i)),
        ],
        out_shape=[
            jax.ShapeDtypeStruct((2, _TOKENS), jnp.float32),
            jax.ShapeDtypeStruct((2, _TOKENS), jnp.int32),
        ],
        compiler_params=pltpu.CompilerParams(
            dimension_semantics=("parallel",),
        ),
    )(x, W, b2)
    return vals_t.T, idx_t.T


# speedup vs baseline: 1.5341x; 1.0113x over previous
import jax
import jax.numpy as jnp
from jax.experimental import pallas as pl
from jax.experimental.pallas import tpu as pltpu

_TOKENS = 32768
_D = 1024
_E = 8
_BT = 2048


def _gate_block(x_ref, w_ref, b_ref, vals_ref, idx_ref):
    x = x_ref[...]
    w = w_ref[...]
    logits = jax.lax.dot_general(
        x, w, (((1,), (1,)), ((), ())), preferred_element_type=jnp.float32)
    lt = logits.T + b_ref[...]

    e = jax.lax.broadcasted_iota(jnp.int32, lt.shape, 0)
    m1 = jnp.max(lt, axis=0, keepdims=True)
    i1 = jnp.min(jnp.where(lt == m1, e, _E), axis=0, keepdims=True)
    masked = jnp.where(e == i1, -jnp.inf, lt)
    m2 = jnp.max(masked, axis=0, keepdims=True)
    i2 = jnp.min(jnp.where(masked == m2, e, _E), axis=0, keepdims=True)

    vals_ref[...] = jax.nn.sigmoid(jnp.concatenate([m1, m2], axis=0))
    idx_ref[...] = jnp.concatenate([i1, i2], axis=0)


def kernel(x, W, b):
    b2 = b.reshape(_E, 1)
    grid = (_TOKENS // _BT,)
    vals_t, idx_t = pl.pallas_call(
        _gate_block,
        grid=grid,
        in_specs=[
            pl.BlockSpec((_BT, _D), lambda i: (i, 0)),
            pl.BlockSpec((_E, _D), lambda i: (0, 0)),
            pl.BlockSpec((_E, 1), lambda i: (0, 0)),
        ],
        out_specs=[
            pl.BlockSpec((2, _BT), lambda i: (0, i)),
            pl.BlockSpec((2, _BT), lambda i: (0, i)),
        ],
        out_shape=[
            jax.ShapeDtypeStruct((2, _TOKENS), jnp.float32),
            jax.ShapeDtypeStruct((2, _TOKENS), jnp.int32),
        ],
        compiler_params=pltpu.CompilerParams(
            dimension_semantics=("parallel",),
        ),
    )(x, W, b2)
    return vals_t.T, idx_t.T
